# baseline (speedup 1.0000x reference)
import functools

import jax
import jax.numpy as jnp
from jax import lax
from jax.experimental import pallas as pl
from jax.experimental.pallas import tpu as pltpu
from jax.experimental.pallas import tpu_sc as plsc

D = 16
B = 16384
NC, NS = 2, 16
NW = NC * NS
BPW = B // NW
WAVE = 4
NWAVES = BPW // WAVE


def _make_kernel():
    mesh = plsc.VectorSubcoreMesh(core_axis_name="c", subcore_axis_name="s")

    @functools.partial(
        pl.kernel,
        out_type=jax.ShapeDtypeStruct((B,), jnp.float32),
        mesh=mesh,
        compiler_params=pltpu.CompilerParams(
            needs_layout_passes=False, use_tc_tiling_on_sc=True),
        scratch_types=[
            pltpu.VMEM((BPW,), jnp.int32),
            pltpu.VMEM((BPW,), jnp.int32),
            pltpu.VMEM((4, WAVE, D, 128), jnp.float32),
            pltpu.VMEM((4, WAVE, D, 128), jnp.float32),
            pltpu.VMEM((16 * D,), jnp.float32),
            pltpu.VMEM((BPW,), jnp.float32),
            pltpu.VMEM((16,), jnp.float32),
            pltpu.SemaphoreType.DMA,
            pltpu.SemaphoreType.DMA,
            pltpu.SemaphoreType.DMA,
            pltpu.SemaphoreType.DMA,
        ],
    )
    def shallow_kernel(rx_hbm, tx_hbm, tbl_hbm, bias_hbm, out_hbm,
                       idx_rx, idx_tx, slabs_a, slabs_b, pane, out_v,
                       bias_v, sem0, sem1, sem2, sem3):
        wid = lax.axis_index("s") * NC + lax.axis_index("c")
        base = wid * BPW

        pltpu.sync_copy(rx_hbm.at[pl.ds(base, BPW)], idx_rx)
        pltpu.sync_copy(tx_hbm.at[pl.ds(base, BPW)], idx_tx)
        pltpu.sync_copy(bias_hbm, bias_v)

        lanes = lax.iota(jnp.int32, 16)
        bias_vec = bias_v[...]

        def scalars_at(ref, b0):
            g16 = (b0 >> 4) << 4
            vec = ref[pl.ds(g16, 16)]
            lb = b0 & 15
            return [
                jnp.max(jnp.where(lanes == lb + l, vec, jnp.int32(-1)))
                for l in range(WAVE)
            ]

        def issue_wave(w, s, sem):
            b0 = w * WAVE
            irs = scalars_at(idx_rx, b0)
            its = scalars_at(idx_tx, b0)
            for l in range(WAVE):
                qr = pl.multiple_of((irs[l] >> 7) << 7, 128)
                qt = pl.multiple_of((its[l] >> 7) << 7, 128)
                for h in range(2):
                    rows = pl.ds(8 * h, 8)
                    pltpu.async_copy(
                        tbl_hbm.at[rows, pl.ds(qr, 128)],
                        slabs_a.at[s, l, rows], sem)
                    pltpu.async_copy(
                        tbl_hbm.at[rows, pl.ds(qt, 128)],
                        slabs_b.at[s, l, rows], sem)

        def drain_wave(s, sem):
            for l in range(WAVE):
                pltpu.make_async_copy(
                    tbl_hbm.at[:, pl.ds(0, 128)], slabs_a.at[s, l], sem).wait()
                pltpu.make_async_copy(
                    tbl_hbm.at[:, pl.ds(0, 128)], slabs_b.at[s, l], sem).wait()

        def compute_wave(w, s):
            b0 = w * WAVE
            quarter = (w % 4) * WAVE
            sl = jnp.full((16,), s, jnp.int32)
            irs = scalars_at(idx_rx, b0)
            its = scalars_at(idx_tx, b0)
            for l in range(WAVE):
                r_r = jnp.full((16,), irs[l] & 127, jnp.int32)
                r_t = jnp.full((16,), its[l] & 127, jnp.int32)
                ll = jnp.full((16,), l, jnp.int32)
                va = plsc.load_gather(slabs_a, [sl, ll, lanes, r_r])
                vb = plsc.load_gather(slabs_b, [sl, ll, lanes, r_t])
                pane[pl.ds((quarter + l) * 16, 16)] = va * vb

        def reduce_pane(w):
            b0 = (w - 3) * WAVE
            acc = jnp.zeros((16,), jnp.float32)
            for j in range(D):
                cidx = ((lanes + j) & 15) + lanes * 16
                acc = acc + plsc.load_gather(pane, [cidx])
            z = acc + bias_vec
            out_v[pl.ds(b0, 16)] = 1.0 / (1.0 + jnp.exp(-z))

        sems = [sem0, sem1, sem2, sem3]
        issue_wave(0, 0, sem0)
        issue_wave(1, 1, sem1)
        issue_wave(2, 2, sem2)

        def step(t, _):
            w_base = 4 * t
            for k in range(4):
                w = w_base + k
                kn = (k + 3) % 4

                @pl.when(w + 3 < NWAVES)
                def _():
                    issue_wave(w + 3, kn, sems[kn])

                drain_wave(k, sems[k])
                compute_wave(w, k)

            reduce_pane(w_base + 3)
            return 0

        lax.fori_loop(0, NWAVES // 4, step, 0)

        pltpu.sync_copy(out_v, out_hbm.at[pl.ds(base, BPW)])

    return shallow_kernel


_shallow = _make_kernel()


def kernel(rx, tx, emb_weight, bias):
    bias16 = jnp.broadcast_to(bias.astype(jnp.float32), (16,))
    return _shallow(rx.astype(jnp.int32), tx.astype(jnp.int32),
                    emb_weight.T, bias16)


# speedup vs baseline: 1.0058x; 1.0058x over previous
import functools

import jax
import jax.numpy as jnp
from jax import lax
from jax.experimental import pallas as pl
from jax.experimental.pallas import tpu as pltpu
from jax.experimental.pallas import tpu_sc as plsc

D = 16
B = 16384
NC, NS = 2, 16
NW = NC * NS
BPW = B // NW
WAVE = 4
NWAVES = BPW // WAVE
NBUF = 4


def _make_kernel():
    mesh = plsc.VectorSubcoreMesh(core_axis_name="c", subcore_axis_name="s")

    @functools.partial(
        pl.kernel,
        out_type=jax.ShapeDtypeStruct((B,), jnp.float32),
        mesh=mesh,
        compiler_params=pltpu.CompilerParams(
            needs_layout_passes=False, use_tc_tiling_on_sc=True),
        scratch_types=[
            pltpu.VMEM((BPW,), jnp.int32),
            pltpu.VMEM((BPW,), jnp.int32),
            pltpu.VMEM((NBUF, WAVE, D, 128), jnp.float32),
            pltpu.VMEM((NBUF, WAVE, D, 128), jnp.float32),
            pltpu.VMEM((16 * D,), jnp.float32),
            pltpu.VMEM((BPW,), jnp.float32),
            pltpu.VMEM((16,), jnp.float32),
            pltpu.SemaphoreType.DMA,
            pltpu.SemaphoreType.DMA,
            pltpu.SemaphoreType.DMA,
            pltpu.SemaphoreType.DMA,
        ],
    )
    def shallow_kernel(rx_hbm, tx_hbm, tbl_hbm, bias_hbm, out_hbm,
                       idx_rx, idx_tx, slabs_a, slabs_b, pane, out_v,
                       bias_v, sem0, sem1, sem2, sem3):
        wid = lax.axis_index("s") * NC + lax.axis_index("c")
        base = wid * BPW

        pltpu.sync_copy(rx_hbm.at[pl.ds(base, BPW)], idx_rx)
        pltpu.sync_copy(tx_hbm.at[pl.ds(base, BPW)], idx_tx)
        pltpu.sync_copy(bias_hbm, bias_v)

        lanes = lax.iota(jnp.int32, 16)
        bias_vec = bias_v[...]

        def scalars_at(ref, b0):
            g16 = (b0 >> 4) << 4
            vec = ref[pl.ds(g16, 16)]
            lb = b0 & 15
            return [
                jnp.max(jnp.where(lanes == lb + l, vec, jnp.int32(-1)))
                for l in range(WAVE)
            ]

        def issue_wave(w, s, sem):
            b0 = w * WAVE
            irs = scalars_at(idx_rx, b0)
            its = scalars_at(idx_tx, b0)
            for l in range(WAVE):
                qr = pl.multiple_of((irs[l] >> 7) << 7, 128)
                qt = pl.multiple_of((its[l] >> 7) << 7, 128)
                pltpu.async_copy(
                    tbl_hbm.at[:, pl.ds(qr, 128)], slabs_a.at[s, l], sem)
                pltpu.async_copy(
                    tbl_hbm.at[:, pl.ds(qt, 128)], slabs_b.at[s, l], sem)

        def drain_wave(s, sem):
            for l in range(WAVE):
                pltpu.make_async_copy(
                    tbl_hbm.at[:, pl.ds(0, 128)], slabs_a.at[s, l], sem).wait()
                pltpu.make_async_copy(
                    tbl_hbm.at[:, pl.ds(0, 128)], slabs_b.at[s, l], sem).wait()

        def compute_wave(w, s):
            b0 = w * WAVE
            quarter = (w % 4) * WAVE
            sl = jnp.full((16,), s, jnp.int32)
            irs = scalars_at(idx_rx, b0)
            its = scalars_at(idx_tx, b0)
            for l in range(WAVE):
                r_r = jnp.full((16,), irs[l] & 127, jnp.int32)
                r_t = jnp.full((16,), its[l] & 127, jnp.int32)
                ll = jnp.full((16,), l, jnp.int32)
                va = plsc.load_gather(slabs_a, [sl, ll, lanes, r_r])
                vb = plsc.load_gather(slabs_b, [sl, ll, lanes, r_t])
                pane[pl.ds((quarter + l) * 16, 16)] = va * vb

        def reduce_pane(w):
            b0 = (w - 3) * WAVE
            acc = jnp.zeros((16,), jnp.float32)
            for j in range(D):
                cidx = ((lanes + j) & 15) + lanes * 16
                acc = acc + plsc.load_gather(pane, [cidx])
            z = acc + bias_vec
            out_v[pl.ds(b0, 16)] = 1.0 / (1.0 + jnp.exp(-z))

        sems = [sem0, sem1, sem2, sem3]
        issue_wave(0, 0, sem0)
        issue_wave(1, 1, sem1)
        issue_wave(2, 2, sem2)

        def step(t, _):
            w_base = NBUF * t
            for k in range(NBUF):
                w = w_base + k
                kn = (k + NBUF - 1) % NBUF

                @pl.when(w + NBUF - 1 < NWAVES)
                def _():
                    issue_wave(w + NBUF - 1, kn, sems[kn])

                drain_wave(k, sems[k])
                compute_wave(w, k)

            reduce_pane(w_base + 3)
            return 0

        lax.fori_loop(0, NWAVES // NBUF, step, 0)

        pltpu.sync_copy(out_v, out_hbm.at[pl.ds(base, BPW)])

    return shallow_kernel


_shallow = _make_kernel()


def kernel(rx, tx, emb_weight, bias):
    bias16 = jnp.broadcast_to(bias.astype(jnp.float32), (16,))
    return _shallow(rx.astype(jnp.int32), tx.astype(jnp.int32),
                    emb_weight.T, bias16)


# speedup vs baseline: 1.0843x; 1.0780x over previous
import functools

import jax
import jax.numpy as jnp
from jax import lax
from jax.experimental import pallas as pl
from jax.experimental.pallas import tpu as pltpu
from jax.experimental.pallas import tpu_sc as plsc

D = 16
B = 16384
NC, NS = 2, 16
NW = NC * NS
BPW = B // NW
WAVE = 2
NWAVES = BPW // WAVE
NBUF = 8


def _make_kernel():
    mesh = plsc.VectorSubcoreMesh(core_axis_name="c", subcore_axis_name="s")

    @functools.partial(
        pl.kernel,
        out_type=jax.ShapeDtypeStruct((B,), jnp.float32),
        mesh=mesh,
        compiler_params=pltpu.CompilerParams(
            needs_layout_passes=False, use_tc_tiling_on_sc=True),
        scratch_types=[
            pltpu.VMEM((BPW,), jnp.int32),
            pltpu.VMEM((BPW,), jnp.int32),
            pltpu.VMEM((NBUF, WAVE, D, 128), jnp.float32),
            pltpu.VMEM((NBUF, WAVE, D, 128), jnp.float32),
            pltpu.VMEM((16 * D,), jnp.float32),
            pltpu.VMEM((BPW,), jnp.float32),
            pltpu.VMEM((16,), jnp.float32),
            pltpu.SemaphoreType.DMA,
            pltpu.SemaphoreType.DMA,
            pltpu.SemaphoreType.DMA,
            pltpu.SemaphoreType.DMA,
            pltpu.SemaphoreType.DMA,
            pltpu.SemaphoreType.DMA,
            pltpu.SemaphoreType.DMA,
            pltpu.SemaphoreType.DMA,
        ],
    )
    def shallow_kernel(rx_hbm, tx_hbm, tbl_hbm, bias_hbm, out_hbm,
                       idx_rx, idx_tx, slabs_a, slabs_b, pane, out_v,
                       bias_v, sem0, sem1, sem2, sem3,
                       sem4, sem5, sem6, sem7):
        wid = lax.axis_index("s") * NC + lax.axis_index("c")
        base = wid * BPW

        pltpu.sync_copy(rx_hbm.at[pl.ds(base, BPW)], idx_rx)
        pltpu.sync_copy(tx_hbm.at[pl.ds(base, BPW)], idx_tx)
        pltpu.sync_copy(bias_hbm, bias_v)

        lanes = lax.iota(jnp.int32, 16)
        bias_vec = bias_v[...]

        def scalars_at(ref, b0):
            g16 = (b0 >> 4) << 4
            vec = ref[pl.ds(g16, 16)]
            lb = b0 & 15
            return [
                jnp.max(jnp.where(lanes == lb + l, vec, jnp.int32(-1)))
                for l in range(WAVE)
            ]

        def issue_wave(w, s, sem):
            b0 = w * WAVE
            irs = scalars_at(idx_rx, b0)
            its = scalars_at(idx_tx, b0)
            for l in range(WAVE):
                qr = pl.multiple_of((irs[l] >> 7) << 7, 128)
                qt = pl.multiple_of((its[l] >> 7) << 7, 128)
                pltpu.async_copy(
                    tbl_hbm.at[:, pl.ds(qr, 128)], slabs_a.at[s, l], sem)
                pltpu.async_copy(
                    tbl_hbm.at[:, pl.ds(qt, 128)], slabs_b.at[s, l], sem)

        def drain_wave(s, sem):
            for l in range(WAVE):
                pltpu.make_async_copy(
                    tbl_hbm.at[:, pl.ds(0, 128)], slabs_a.at[s, l], sem).wait()
                pltpu.make_async_copy(
                    tbl_hbm.at[:, pl.ds(0, 128)], slabs_b.at[s, l], sem).wait()

        def compute_wave(w, s):
            b0 = w * WAVE
            quarter = (w % 8) * WAVE
            sl = jnp.full((16,), s, jnp.int32)
            irs = scalars_at(idx_rx, b0)
            its = scalars_at(idx_tx, b0)
            for l in range(WAVE):
                r_r = jnp.full((16,), irs[l] & 127, jnp.int32)
                r_t = jnp.full((16,), its[l] & 127, jnp.int32)
                ll = jnp.full((16,), l, jnp.int32)
                va = plsc.load_gather(slabs_a, [sl, ll, lanes, r_r])
                vb = plsc.load_gather(slabs_b, [sl, ll, lanes, r_t])
                pane[pl.ds((quarter + l) * 16, 16)] = va * vb

        def reduce_pane(w):
            b0 = (w - 7) * WAVE
            acc = jnp.zeros((16,), jnp.float32)
            for j in range(D):
                cidx = ((lanes + j) & 15) + lanes * 16
                acc = acc + plsc.load_gather(pane, [cidx])
            z = acc + bias_vec
            out_v[pl.ds(b0, 16)] = 1.0 / (1.0 + jnp.exp(-z))

        sems = [sem0, sem1, sem2, sem3, sem4, sem5, sem6, sem7]
        for p in range(NBUF - 1):
            issue_wave(p, p, sems[p])

        def step(t, _):
            w_base = NBUF * t
            for k in range(NBUF):
                w = w_base + k
                kn = (k + NBUF - 1) % NBUF

                @pl.when(w + NBUF - 1 < NWAVES)
                def _():
                    issue_wave(w + NBUF - 1, kn, sems[kn])

                drain_wave(k, sems[k])
                compute_wave(w, k)

            reduce_pane(w_base + NBUF - 1)
            return 0

        lax.fori_loop(0, NWAVES // NBUF, step, 0)

        pltpu.sync_copy(out_v, out_hbm.at[pl.ds(base, BPW)])

    return shallow_kernel


_shallow = _make_kernel()


def kernel(rx, tx, emb_weight, bias):
    bias16 = jnp.broadcast_to(bias.astype(jnp.float32), (16,))
    return _shallow(rx.astype(jnp.int32), tx.astype(jnp.int32),
                    emb_weight.T, bias16)
